# baseline (speedup 1.0000x reference)
import jax
import jax.numpy as jnp
from jax.experimental import pallas as pl
from jax.experimental.pallas import tpu as pltpu

_N_STEPS = 4


def _make_body(batch, seq, d):
    rows = seq // _N_STEPS

    def body(w_ref, out_hbm, sems):
        i = pl.program_id(0)
        sl = pl.ds(i * rows, rows)
        cps = []
        for b in range(batch):
            cp = pltpu.make_async_copy(w_ref, out_hbm.at[b, sl, :], sems.at[b])
            cp.start()
            cps.append(cp)
        for cp in cps:
            cp.wait()

    return body


def kernel(tokens, W_pos):
    batch, seq = tokens.shape
    d = W_pos.shape[-1]
    rows = seq // _N_STEPS
    return pl.pallas_call(
        _make_body(batch, seq, d),
        grid=(_N_STEPS,),
        in_specs=[pl.BlockSpec((rows, d), lambda i: (i, 0))],
        out_specs=pl.BlockSpec(memory_space=pltpu.MemorySpace.HBM),
        out_shape=jax.ShapeDtypeStruct((batch, seq, d), W_pos.dtype),
        scratch_shapes=[
            pltpu.SemaphoreType.DMA((batch,)),
        ],
    )(W_pos[:seq])


# speedup vs baseline: 1.3703x; 1.3703x over previous
import jax
import jax.numpy as jnp
from jax.experimental import pallas as pl
from jax.experimental.pallas import tpu as pltpu

_CHUNK_FRACS = (1, 1, 2, 2, 2, 2, 3, 3)


def _make_body(batch, seq, d, bounds):
    n_chunks = len(bounds) - 1

    def body(w_hbm, out_hbm, w_vmem, in_sems, out_sems):
        in_cps = []
        for c in range(n_chunks):
            sl = pl.ds(bounds[c], bounds[c + 1] - bounds[c])
            cp = pltpu.make_async_copy(
                w_hbm.at[sl, :], w_vmem.at[sl, :], in_sems.at[c])
            cp.start()
            in_cps.append(cp)
        out_cps = []
        for c in range(n_chunks):
            in_cps[c].wait()
            sl = pl.ds(bounds[c], bounds[c + 1] - bounds[c])
            for b in range(batch):
                cp = pltpu.make_async_copy(
                    w_vmem.at[sl, :], out_hbm.at[b, sl, :], out_sems.at[b, c])
                cp.start()
                out_cps.append(cp)
        for cp in out_cps:
            cp.wait()

    return body


def kernel(tokens, W_pos):
    batch, seq = tokens.shape
    d = W_pos.shape[-1]
    total = sum(_CHUNK_FRACS)
    bounds = [0]
    for f in _CHUNK_FRACS:
        bounds.append(bounds[-1] + seq * f // total)
    bounds[-1] = seq
    return pl.pallas_call(
        _make_body(batch, seq, d, bounds),
        in_specs=[pl.BlockSpec(memory_space=pltpu.MemorySpace.HBM)],
        out_specs=pl.BlockSpec(memory_space=pltpu.MemorySpace.HBM),
        out_shape=jax.ShapeDtypeStruct((batch, seq, d), W_pos.dtype),
        scratch_shapes=[
            pltpu.VMEM((seq, d), W_pos.dtype),
            pltpu.SemaphoreType.DMA((len(_CHUNK_FRACS),)),
            pltpu.SemaphoreType.DMA((batch, len(_CHUNK_FRACS))),
        ],
    )(W_pos[:seq])


# speedup vs baseline: 1.4096x; 1.0287x over previous
import jax
import jax.numpy as jnp
from jax.experimental import pallas as pl
from jax.experimental.pallas import tpu as pltpu

_CHUNK_FRACS = (1, 3, 5, 7)


def _make_body(batch, seq, d, bounds):
    n_chunks = len(bounds) - 1

    def body(w_hbm, out_hbm, w_vmem, in_sems, out_sems):
        in_cps = []
        for c in range(n_chunks):
            sl = pl.ds(bounds[c], bounds[c + 1] - bounds[c])
            cp = pltpu.make_async_copy(
                w_hbm.at[sl, :], w_vmem.at[sl, :], in_sems.at[c])
            cp.start()
            in_cps.append(cp)
        out_cps = []
        for c in range(n_chunks):
            in_cps[c].wait()
            sl = pl.ds(bounds[c], bounds[c + 1] - bounds[c])
            for b in range(batch):
                cp = pltpu.make_async_copy(
                    w_vmem.at[sl, :], out_hbm.at[b, sl, :], out_sems.at[b, c])
                cp.start()
                out_cps.append(cp)
        for cp in out_cps:
            cp.wait()

    return body


def kernel(tokens, W_pos):
    batch, seq = tokens.shape
    d = W_pos.shape[-1]
    total = sum(_CHUNK_FRACS)
    bounds = [0]
    for f in _CHUNK_FRACS:
        bounds.append(bounds[-1] + seq * f // total)
    bounds[-1] = seq
    return pl.pallas_call(
        _make_body(batch, seq, d, bounds),
        in_specs=[pl.BlockSpec(memory_space=pltpu.MemorySpace.HBM)],
        out_specs=pl.BlockSpec(memory_space=pltpu.MemorySpace.HBM),
        out_shape=jax.ShapeDtypeStruct((batch, seq, d), W_pos.dtype),
        scratch_shapes=[
            pltpu.VMEM((seq, d), W_pos.dtype),
            pltpu.SemaphoreType.DMA((len(_CHUNK_FRACS),)),
            pltpu.SemaphoreType.DMA((batch, len(_CHUNK_FRACS))),
        ],
    )(W_pos[:seq])


# speedup vs baseline: 1.4332x; 1.0167x over previous
import jax
import jax.numpy as jnp
from jax.experimental import pallas as pl
from jax.experimental.pallas import tpu as pltpu

_CHUNK_FRACS = (1, 3, 12)


def _make_body(batch, seq, d, bounds):
    n_chunks = len(bounds) - 1

    def body(w_hbm, out_hbm, w_vmem, in_sems, out_sems):
        in_cps = []
        for c in range(n_chunks):
            sl = pl.ds(bounds[c], bounds[c + 1] - bounds[c])
            cp = pltpu.make_async_copy(
                w_hbm.at[sl, :], w_vmem.at[sl, :], in_sems.at[c])
            cp.start()
            in_cps.append(cp)
        out_cps = []
        for c in range(n_chunks):
            in_cps[c].wait()
            sl = pl.ds(bounds[c], bounds[c + 1] - bounds[c])
            for b in range(batch):
                cp = pltpu.make_async_copy(
                    w_vmem.at[sl, :], out_hbm.at[b, sl, :], out_sems.at[b, c])
                cp.start()
                out_cps.append(cp)
        for cp in out_cps:
            cp.wait()

    return body


def kernel(tokens, W_pos):
    batch, seq = tokens.shape
    d = W_pos.shape[-1]
    total = sum(_CHUNK_FRACS)
    bounds = [0]
    for f in _CHUNK_FRACS:
        bounds.append(bounds[-1] + seq * f // total)
    bounds[-1] = seq
    return pl.pallas_call(
        _make_body(batch, seq, d, bounds),
        in_specs=[pl.BlockSpec(memory_space=pltpu.MemorySpace.HBM)],
        out_specs=pl.BlockSpec(memory_space=pltpu.MemorySpace.HBM),
        out_shape=jax.ShapeDtypeStruct((batch, seq, d), W_pos.dtype),
        scratch_shapes=[
            pltpu.VMEM((seq, d), W_pos.dtype),
            pltpu.SemaphoreType.DMA((len(_CHUNK_FRACS),)),
            pltpu.SemaphoreType.DMA((batch, len(_CHUNK_FRACS))),
        ],
    )(W_pos[:seq])
